# baseline (speedup 1.0000x reference)
import functools

import jax
import jax.numpy as jnp
from jax import lax
from jax.experimental import pallas as pl
from jax.experimental.pallas import tpu as pltpu
from jax.experimental.pallas import tpu_sc as plsc

D = 32
CH = 3200
NC = 2
NS = 16
NW = NC * NS

CBLK = 10000


def _combine_body(a_ref, b_ref, o_ref):
    o_ref[...] = a_ref[...] + b_ref[...]


def _combine(token_table, pos_table):
    V = token_table.shape[0]
    R = V * D // 128
    t = token_table.reshape(R, 128)
    p = pos_table.reshape(R, 128)
    out = pl.pallas_call(
        _combine_body,
        grid=(R // CBLK,),
        in_specs=[
            pl.BlockSpec((CBLK, 128), lambda i: (i, 0)),
            pl.BlockSpec((CBLK, 128), lambda i: (i, 0)),
        ],
        out_specs=pl.BlockSpec((CBLK, 128), lambda i: (i, 0)),
        out_shape=jax.ShapeDtypeStruct((R, 128), jnp.float32),
    )(t, p)
    return out.reshape(V, D)


@functools.partial(jax.jit, static_argnums=(0,))
def _lookup_add(B, idx_flat, token_table, pos_table):
    combined = _combine(token_table, pos_table)

    b_per_w = B // NW
    n_chunks = b_per_w // CH
    mesh = plsc.VectorSubcoreMesh(core_axis_name="c", subcore_axis_name="s")

    @functools.partial(
        pl.kernel,
        out_type=jax.ShapeDtypeStruct((B, D), jnp.float32),
        mesh=mesh,
        compiler_params=pltpu.CompilerParams(use_tc_tiling_on_sc=False),
        scratch_types=[
            pltpu.VMEM((CH,), jnp.int32),
            pltpu.VMEM((CH, D), jnp.float32),
            pltpu.SemaphoreType.DMA,
        ],
    )
    def k(idx_hbm, tab_hbm, out_hbm, idx_v, buf, sem):
        wid = lax.axis_index("s") * NC + lax.axis_index("c")
        w_base = wid * b_per_w

        def chunk_body(c, carry):
            base = pl.multiple_of(w_base + c * CH, CH)
            pltpu.sync_copy(idx_hbm.at[pl.ds(base, CH)], idx_v)
            pltpu.async_copy(tab_hbm.at[idx_v], buf, sem).wait()
            pltpu.sync_copy(buf, out_hbm.at[pl.ds(base, CH)])
            return carry

        lax.fori_loop(0, n_chunks, chunk_body, 0)

    return k(idx_flat, combined)


def kernel(x, token_table, pos_table):
    batch, seq = x.shape
    B = batch * seq
    out = _lookup_add(B, x.reshape(B), token_table, pos_table)
    return out.reshape(batch, seq, D)
